# Initial kernel scaffold; baseline (speedup 1.0000x reference)
import jax
import jax.numpy as jnp
from jax.experimental import pallas as pl


def kernel(x, edge_index, edge_attr, batch, global_features, shap_embedding, params):
    raise NotImplementedError("write your pallas kernel here")


# speedup vs baseline: 15.5050x; 15.5050x over previous
import functools

import jax
import jax.numpy as jnp
from jax import lax
from jax.experimental import pallas as pl
from jax.experimental.pallas import tpu as pltpu
from jax.experimental.pallas import tpu_sc as plsc

N = 10000
E = 320000
B = 64
H = 8
C = 16
F = H * C

NE = E + N
NT = 32
CH = 64
EP = 331776
EPT = EP // NT
NCH = EPT // CH
NP = 10240
NPT = NP // 16
ND = NP // 8
NDT = ND // 16

_mesh = plsc.VectorSubcoreMesh(core_axis_name="c", subcore_axis_name="s")
_f32 = jnp.float32
_i32 = jnp.int32

_GDN = lax.GatherDimensionNumbers(
    offset_dims=(), collapsed_slice_dims=(0,), start_index_map=(0,))


def _perm(v, idx):
    return lax.gather(v, idx[:, None], _GDN, (1,),
                      mode=lax.GatherScatterMode.PROMISE_IN_BOUNDS)


def _bcast_lane(v, lane):
    return _perm(v, jnp.full((16,), lane, _i32))


def _lanesum(v, rots):
    for r in rots:
        v = v + _perm(v, r)
    return v


@functools.partial(
    pl.kernel,
    out_type=[
        jax.ShapeDtypeStruct((EP, 16), _f32),
        jax.ShapeDtypeStruct((2 * ND, F), _f32),
    ],
    mesh=_mesh,
    scratch_types=[
        pltpu.VMEM((CH,), _i32),
        pltpu.VMEM((CH,), _i32),
        pltpu.VMEM((CH,), _i32),
        pltpu.VMEM((CH,), _f32),
        pltpu.VMEM((CH,), _f32),
        pltpu.VMEM((H, 16), _f32),
        pltpu.VMEM((H, 16), _f32),
        pltpu.VMEM((H, 16), _f32),
        pltpu.VMEM((16, 16), _f32),
        pltpu.VMEM((CH, F), _f32),
        pltpu.VMEM((CH, F), _f32),
        pltpu.VMEM((CH, 16), _f32),
        pltpu.VMEM((CH, F), _f32),
        pltpu.VMEM_SHARED((ND, F), _f32),
        pltpu.SemaphoreType.DMA,
        pltpu.SemaphoreType.DMA,
    ],
)
def _sc_logits(hl, hr, s_hbm, d_hbm, d8_hbm, ea_hbm, dmf_hbm, web_hbm,
               attb_hbm, oh_hbm, kh_hbm, zd_hbm,
               ex_hbm, den_hbm,
               s_ch, d_ch, d8_ch, ea_ch, dmf_ch, web_vm, attb_vm, oh_vm,
               kh_vm, sbuf, rbuf, ex_st, exf_st, den_sh, sem_g, sem_s):
    ci = lax.axis_index("c")
    si = lax.axis_index("s")
    wid = ci * 16 + si
    base = wid * EPT
    pltpu.sync_copy(web_hbm, web_vm)
    pltpu.sync_copy(attb_hbm, attb_vm)
    pltpu.sync_copy(oh_hbm, oh_vm)
    pltpu.sync_copy(kh_hbm, kh_vm)
    pltpu.sync_copy(zd_hbm, den_sh.at[pl.ds(si * NDT, NDT)])
    plsc.subcore_barrier()

    it16 = lax.iota(_i32, 16)
    rots = [(it16 + r) & 15 for r in (8, 4, 2, 1)]
    webs = [web_vm[h] for h in range(H)]
    atts = [attb_vm[h] for h in range(H)]
    ohs = [oh_vm[h] for h in range(H)]
    khdr = kh_vm[0]

    def chunk_body(cc, _):
        off = base + cc * CH
        pltpu.sync_copy(s_hbm.at[pl.ds(off, CH)], s_ch)
        pltpu.sync_copy(d_hbm.at[pl.ds(off, CH)], d_ch)
        pltpu.sync_copy(d8_hbm.at[pl.ds(off, CH)], d8_ch)
        pltpu.sync_copy(ea_hbm.at[pl.ds(off, CH)], ea_ch)
        pltpu.sync_copy(dmf_hbm.at[pl.ds(off, CH)], dmf_ch)
        pltpu.async_copy(hl.at[s_ch], sbuf, sem_g).wait()
        pltpu.async_copy(hr.at[d_ch], rbuf, sem_g).wait()

        def gbody(g, _g):
            eav = ea_ch[pl.ds(g * 16, 16)]
            dmfv = dmf_ch[pl.ds(g * 16, 16)]

            def ebody(j, _e):
                e = g * 16 + j
                eab = _bcast_lane(eav, j)
                dmb = _bcast_lane(dmfv, j)
                acc = jnp.zeros((16,), _f32)
                for h in range(H):
                    u = (sbuf[e, pl.ds(h * 16, 16)]
                         + rbuf[e, pl.ds(h * 16, 16)] + eab * webs[h])
                    u = jnp.maximum(u, 0.2 * u)
                    tot = _lanesum(u * atts[h], rots)
                    acc = acc + tot * ohs[h]
                exv = jnp.exp(acc)
                livef = jnp.where((off + e) < NE, 1.0, 0.0)
                keep = khdr * lax.broadcast_in_dim(livef, (16,), ())
                exv = exv * keep
                ex_st[e] = exv
                for j8 in range(8):
                    msk = jnp.maximum(0.0, 1.0 - jnp.abs(dmb - float(j8)))
                    exf_st[e, pl.ds(j8 * 16, 16)] = exv * msk
                return 0

            lax.fori_loop(0, 16, ebody, 0)
            return 0

        lax.fori_loop(0, CH // 16, gbody, 0)
        pltpu.sync_copy(ex_st, ex_hbm.at[pl.ds(off, CH)])
        pltpu.async_copy(exf_st, den_sh.at[d8_ch], sem_s, add=True).wait()
        return 0

    lax.fori_loop(0, NCH, chunk_body, 0)
    plsc.subcore_barrier()
    pltpu.sync_copy(den_sh.at[pl.ds(si * NDT, NDT)],
                    den_hbm.at[pl.ds(ci * ND + si * NDT, NDT)])


@functools.partial(
    pl.kernel,
    out_type=[jax.ShapeDtypeStruct((2 * NP, F), _f32)],
    mesh=_mesh,
    scratch_types=[
        pltpu.VMEM((CH,), _i32),
        pltpu.VMEM((CH,), _i32),
        pltpu.VMEM((CH,), _i32),
        pltpu.VMEM((CH,), _f32),
        pltpu.VMEM((CH, F), _f32),
        pltpu.VMEM((CH, 16), _f32),
        pltpu.VMEM((CH, F), _f32),
        pltpu.VMEM((CH, F), _f32),
        pltpu.VMEM_SHARED((NP, F), _f32),
        pltpu.SemaphoreType.DMA,
        pltpu.SemaphoreType.DMA,
    ],
)
def _sc_aggregate(hl, ex_hbm, dinv_hbm, s_hbm, d_hbm, d8_hbm, dmf_hbm,
                  zf_hbm,
                  out_hbm,
                  s_ch, d_ch, d8_ch, dmf_ch, sbuf, ex_st, dvb, mbuf, out_sh,
                  sem_g, sem_s):
    ci = lax.axis_index("c")
    si = lax.axis_index("s")
    wid = ci * 16 + si
    base = wid * EPT
    pltpu.sync_copy(zf_hbm, out_sh.at[pl.ds(si * NPT, NPT)])
    plsc.subcore_barrier()

    def chunk_body(cc, _):
        off = base + cc * CH
        pltpu.sync_copy(s_hbm.at[pl.ds(off, CH)], s_ch)
        pltpu.sync_copy(d_hbm.at[pl.ds(off, CH)], d_ch)
        pltpu.sync_copy(d8_hbm.at[pl.ds(off, CH)], d8_ch)
        pltpu.sync_copy(dmf_hbm.at[pl.ds(off, CH)], dmf_ch)
        pltpu.sync_copy(ex_hbm.at[pl.ds(off, CH)], ex_st)
        pltpu.async_copy(hl.at[s_ch], sbuf, sem_g).wait()
        pltpu.async_copy(dinv_hbm.at[d8_ch], dvb, sem_g).wait()

        def gbody(g, _g):
            dmfv = dmf_ch[pl.ds(g * 16, 16)]

            def ebody(j, _e):
                e = g * 16 + j
                dmb = _bcast_lane(dmfv, j)
                dv16 = jnp.zeros((16,), _f32)
                for j8 in range(8):
                    msk = jnp.maximum(0.0, 1.0 - jnp.abs(dmb - float(j8)))
                    dv16 = dv16 + dvb[e, pl.ds(j8 * 16, 16)] * msk
                w = ex_st[e] * dv16
                for h in range(H):
                    wb = _bcast_lane(w, h)
                    mbuf[e, pl.ds(h * 16, 16)] = (
                        wb * sbuf[e, pl.ds(h * 16, 16)])
                return 0

            lax.fori_loop(0, 16, ebody, 0)
            return 0

        lax.fori_loop(0, CH // 16, gbody, 0)
        pltpu.async_copy(mbuf, out_sh.at[d_ch], sem_s, add=True).wait()
        return 0

    lax.fori_loop(0, NCH, chunk_body, 0)
    plsc.subcore_barrier()
    pltpu.sync_copy(out_sh.at[pl.ds(si * NPT, NPT)],
                    out_hbm.at[pl.ds(ci * NP + si * NPT, NPT)])


def _dot(a, b):
    return jnp.dot(a, b, precision=lax.Precision.HIGHEST,
                   preferred_element_type=_f32)


def _prep_body(x_ref, ea_ref, wne_ref, bne_ref, wl_ref, bl_ref, wr_ref,
               br_ref, ea2_ref, hl_ref, hr_ref):
    h0 = jax.nn.relu(_dot(x_ref[...], wne_ref[...]) + bne_ref[...])
    hl_ref[...] = _dot(h0, wl_ref[...]) + bl_ref[...]
    hr_ref[...] = _dot(h0, wr_ref[...]) + br_ref[...]
    mean = jnp.sum(ea_ref[...]) / E
    ea2_ref[0:2500, :] = ea_ref[...]
    gid = (E + lax.broadcasted_iota(_i32, (92, F), 0) * F
           + lax.broadcasted_iota(_i32, (92, F), 1))
    ea2_ref[2500:2592, :] = jnp.where(gid < NE, mean, 0.0)


_prep = pl.pallas_call(
    _prep_body,
    out_shape=[
        jax.ShapeDtypeStruct((2592, F), _f32),
        jax.ShapeDtypeStruct((N, F), _f32),
        jax.ShapeDtypeStruct((N, F), _f32),
    ],
)


def _mid_body(den_ref, dinv_ref):
    dinv_ref[...] = 1.0 / (den_ref[0] + den_ref[1] + 1e-16)


_mid = pl.pallas_call(
    _mid_body,
    out_shape=[jax.ShapeDtypeStruct((ND, F), _f32)],
)


def _post_body(has_resid, has_next, *refs):
    if has_resid:
        (op_ref, bias_ref, g_ref, b_ref, res_ref, *rest) = refs
    else:
        (op_ref, bias_ref, g_ref, b_ref, *rest) = refs
    if has_next:
        (wl_ref, bl_ref, wr_ref, br_ref, h_ref, hl_ref, hr_ref) = rest
    else:
        (h_ref,) = rest
    y = op_ref[0, 0:N, :] + op_ref[1, 0:N, :] + bias_ref[...]
    mu = jnp.mean(y, axis=0)
    d = y - mu
    var = jnp.mean(d * d, axis=0)
    z = d * lax.rsqrt(var + 1e-5) * g_ref[...] + b_ref[...]
    if has_resid:
        z = z + res_ref[...]
    h = jax.nn.relu(z)
    h_ref[...] = h
    if has_next:
        hl_ref[...] = _dot(h, wl_ref[...]) + bl_ref[...]
        hr_ref[...] = _dot(h, wr_ref[...]) + br_ref[...]


def _make_post(has_resid, has_next):
    outs = [jax.ShapeDtypeStruct((N, F), _f32)]
    if has_next:
        outs = outs + [jax.ShapeDtypeStruct((N, F), _f32),
                       jax.ShapeDtypeStruct((N, F), _f32)]
    return pl.pallas_call(
        functools.partial(_post_body, has_resid, has_next), out_shape=outs)


_post_first = _make_post(False, True)
_post_midl = _make_post(True, True)
_post_last = _make_post(True, False)


def _head_body(h_ref, batch_ref, m_in_ref, g_in_ref, p_in_ref, shap_ref,
               wm1, bm1, wm2, bm2, wg1, bg1, wg2, bg2, wp1, bp1, wp2, bp2,
               ws, bs, w1p, w1m, w1g, w1pc, w1s, b1, w2, b2, w3, b3,
               out_ref):
    onehot = jnp.where(
        lax.broadcasted_iota(_i32, (B, N), 0) == batch_ref[...], 1.0, 0.0)
    cnt = jnp.sum(onehot, axis=1, keepdims=True)
    pooled = _dot(onehot, h_ref[...]) / jnp.maximum(cnt, 1.0)
    meta = jax.nn.relu(_dot(jax.nn.relu(
        _dot(m_in_ref[...], wm1[...]) + bm1[...]), wm2[...]) + bm2[...])
    gre = jax.nn.relu(_dot(jax.nn.relu(
        _dot(g_in_ref[...], wg1[...]) + bg1[...]), wg2[...]) + bg2[...])
    pca = jax.nn.relu(_dot(jax.nn.relu(
        _dot(p_in_ref[...], wp1[...]) + bp1[...]), wp2[...]) + bp2[...])
    xs = jax.nn.relu(_dot(shap_ref[...], ws[...]) + bs[...])
    z = (_dot(pooled, w1p[...]) + _dot(meta, w1m[...]) + _dot(gre, w1g[...])
         + _dot(pca, w1pc[...]) + _dot(xs, w1s[...]) + b1[...])
    z = jax.nn.relu(z)
    z = jax.nn.relu(_dot(z, w2[...]) + b2[...])
    out_ref[...] = _dot(z, w3[...]) + b3[...]


_head = pl.pallas_call(
    _head_body, out_shape=[jax.ShapeDtypeStruct((B, 1), _f32)])


def _gat_layer(hl, hr, edges, p, consts):
    s_all, d_all, d8_all, ea2, dmf_all = edges
    oh, kh, zd, zf = consts
    we_r = p["We"][0].reshape(H, 16)
    att_r = p["att"]
    ex, den = _sc_logits(hl, hr, s_all, d_all, d8_all, ea2, dmf_all,
                         we_r, att_r, oh, kh, zd)
    dinv, = _mid(den.reshape(2, ND, F))
    out_parts, = _sc_aggregate(hl, ex, dinv, s_all, d_all, d8_all, dmf_all,
                               zf)
    return out_parts.reshape(2, NP, F)


def kernel(x, edge_index, edge_attr, batch, global_features, shap_embedding,
           params):
    p = params
    src = edge_index[0].astype(_i32)
    dst = edge_index[1].astype(_i32)
    loop = jnp.arange(N, dtype=_i32)
    padz = jnp.zeros((EP - NE,), _i32)
    s_all = jnp.concatenate([src, loop, padz])
    d_all = jnp.concatenate([dst, loop, padz])
    d8_all = d_all // 8
    dmf_all = (d_all % 8).astype(_f32)
    ea_r = edge_attr[:, 0].reshape(2500, F)
    zd = jnp.zeros((NDT, F), _f32)
    zf = jnp.zeros((NPT, F), _f32)
    oh = jnp.eye(H, 16, dtype=_f32)
    kh = jnp.broadcast_to(
        (jnp.arange(16) < H).astype(_f32)[None, :], (16, 16))

    ea2_r, hl, hr = _prep(x, ea_r, p["ne"]["W"], p["ne"]["b"],
                          p["g1"]["Wl"], p["g1"]["bl"],
                          p["g1"]["Wr"], p["g1"]["br"])
    ea2 = ea2_r.reshape(EP)
    edges = (s_all, d_all, d8_all, ea2, dmf_all)
    consts = (oh, kh, zd, zf)

    op = _gat_layer(hl, hr, edges, p["g1"], consts)
    h1, hl, hr = _post_first(op, p["g1"]["bias"], p["bn1"]["g"],
                             p["bn1"]["b"],
                             p["g2"]["Wl"], p["g2"]["bl"],
                             p["g2"]["Wr"], p["g2"]["br"])
    op = _gat_layer(hl, hr, edges, p["g2"], consts)
    h2, hl, hr = _post_midl(op, p["g2"]["bias"], p["bn2"]["g"], p["bn2"]["b"],
                            h1, p["g3"]["Wl"], p["g3"]["bl"],
                            p["g3"]["Wr"], p["g3"]["br"])
    op = _gat_layer(hl, hr, edges, p["g3"], consts)
    h3, hl, hr = _post_midl(op, p["g3"]["bias"], p["bn3"]["g"], p["bn3"]["b"],
                            h2, p["g4"]["Wl"], p["g4"]["bl"],
                            p["g4"]["Wr"], p["g4"]["br"])
    op = _gat_layer(hl, hr, edges, p["g4"], consts)
    h4, = _post_last(op, p["g4"]["bias"], p["bn4"]["g"], p["bn4"]["b"], h3)

    gf = global_features[:, 0, :]
    w1 = p["fc1"]["W"]
    out, = _head(h4, batch.astype(_i32).reshape(1, N),
                 gf[:, 0:4], gf[:, 4:6], gf[:, 6:16], shap_embedding,
                 p["meta1"]["W"], p["meta1"]["b"], p["meta2"]["W"],
                 p["meta2"]["b"], p["gr1"]["W"], p["gr1"]["b"],
                 p["gr2"]["W"], p["gr2"]["b"], p["pca1"]["W"], p["pca1"]["b"],
                 p["pca2"]["W"], p["pca2"]["b"], p["shap"]["W"],
                 p["shap"]["b"], w1[0:128], w1[128:144], w1[144:160],
                 w1[160:192], w1[192:224], p["fc1"]["b"],
                 p["fc2"]["W"], p["fc2"]["b"], p["fc3"]["W"], p["fc3"]["b"])
    return out


# speedup vs baseline: 16.8687x; 1.0880x over previous
import functools

import jax
import jax.numpy as jnp
from jax import lax
from jax.experimental import pallas as pl
from jax.experimental.pallas import tpu as pltpu
from jax.experimental.pallas import tpu_sc as plsc

N = 10000
E = 320000
B = 64
H = 8
C = 16
F = H * C

NE = E + N
NT = 32
CH = 64
EP = 331776
EPT = EP // NT
NCH = EPT // CH
CHA = 128
NCHA = EPT // CHA
NP = 10240
NPT = NP // 16
ND = NP // 8
NDT = ND // 16

_mesh = plsc.VectorSubcoreMesh(core_axis_name="c", subcore_axis_name="s")
_f32 = jnp.float32
_i32 = jnp.int32

_GDN = lax.GatherDimensionNumbers(
    offset_dims=(), collapsed_slice_dims=(0,), start_index_map=(0,))


def _perm(v, idx):
    return lax.gather(v, idx[:, None], _GDN, (1,),
                      mode=lax.GatherScatterMode.PROMISE_IN_BOUNDS)


def _bcast_lane(v, lane):
    return _perm(v, jnp.full((16,), lane, _i32))


def _lanesum(v, rots):
    for r in rots:
        v = v + _perm(v, r)
    return v


@functools.partial(
    pl.kernel,
    out_type=[
        jax.ShapeDtypeStruct((EP, 16), _f32),
        jax.ShapeDtypeStruct((2 * ND, F), _f32),
    ],
    mesh=_mesh,
    scratch_types=[
        pltpu.VMEM((CHA,), _i32),
        pltpu.VMEM((CHA,), _i32),
        pltpu.VMEM((CHA,), _i32),
        pltpu.VMEM((CHA,), _f32),
        pltpu.VMEM((CHA,), _f32),
        pltpu.VMEM((H, 16), _f32),
        pltpu.VMEM((H, 16), _f32),
        pltpu.VMEM((H, 16), _f32),
        pltpu.VMEM((16, 16), _f32),
        pltpu.VMEM((CHA, F), _f32),
        pltpu.VMEM((CHA, F), _f32),
        pltpu.VMEM((CHA, 16), _f32),
        pltpu.VMEM((CHA, F), _f32),
        pltpu.VMEM_SHARED((ND, F), _f32),
        pltpu.SemaphoreType.DMA,
        pltpu.SemaphoreType.DMA,
    ],
)
def _sc_logits(hl, hr, s_hbm, d_hbm, d8_hbm, ea_hbm, dmf_hbm, web_hbm,
               attb_hbm, oh_hbm, kh_hbm, zd_hbm,
               ex_hbm, den_hbm,
               s_ch, d_ch, d8_ch, ea_ch, dmf_ch, web_vm, attb_vm, oh_vm,
               kh_vm, sbuf, rbuf, ex_st, exf_st, den_sh, sem_g, sem_s):
    ci = lax.axis_index("c")
    si = lax.axis_index("s")
    wid = ci * 16 + si
    base = wid * EPT
    pltpu.sync_copy(web_hbm, web_vm)
    pltpu.sync_copy(attb_hbm, attb_vm)
    pltpu.sync_copy(oh_hbm, oh_vm)
    pltpu.sync_copy(kh_hbm, kh_vm)
    pltpu.sync_copy(zd_hbm, den_sh.at[pl.ds(si * NDT, NDT)])
    plsc.subcore_barrier()

    it16 = lax.iota(_i32, 16)
    rots = [(it16 + r) & 15 for r in (8, 4, 2, 1)]
    webs = [web_vm[h] for h in range(H)]
    atts = [attb_vm[h] for h in range(H)]
    ohs = [oh_vm[h] for h in range(H)]
    khdr = kh_vm[0]

    def chunk_body(cc, _):
        off = base + cc * CHA
        pltpu.sync_copy(s_hbm.at[pl.ds(off, CHA)], s_ch)
        pltpu.sync_copy(d_hbm.at[pl.ds(off, CHA)], d_ch)
        pltpu.sync_copy(d8_hbm.at[pl.ds(off, CHA)], d8_ch)
        pltpu.sync_copy(ea_hbm.at[pl.ds(off, CHA)], ea_ch)
        pltpu.sync_copy(dmf_hbm.at[pl.ds(off, CHA)], dmf_ch)
        pltpu.async_copy(hl.at[s_ch], sbuf, sem_g).wait()
        pltpu.async_copy(hr.at[d_ch], rbuf, sem_g).wait()

        def gbody(g, _g):
            eav = ea_ch[pl.ds(g * 16, 16)]
            dmfv = dmf_ch[pl.ds(g * 16, 16)]

            def ebody(j, _e):
                e = g * 16 + j
                eab = _bcast_lane(eav, j)
                dmb = _bcast_lane(dmfv, j)
                acc = jnp.zeros((16,), _f32)
                for h in range(H):
                    u = (sbuf[e, pl.ds(h * 16, 16)]
                         + rbuf[e, pl.ds(h * 16, 16)] + eab * webs[h])
                    u = jnp.maximum(u, 0.2 * u)
                    tot = _lanesum(u * atts[h], rots)
                    acc = acc + tot * ohs[h]
                exv = jnp.exp(acc)
                livef = jnp.where((off + e) < NE, 1.0, 0.0)
                keep = khdr * lax.broadcast_in_dim(livef, (16,), ())
                exv = exv * keep
                ex_st[e] = exv
                for j8 in range(8):
                    msk = jnp.maximum(0.0, 1.0 - jnp.abs(dmb - float(j8)))
                    exf_st[e, pl.ds(j8 * 16, 16)] = exv * msk
                return 0

            lax.fori_loop(0, 16, ebody, 0)
            return 0

        lax.fori_loop(0, CHA // 16, gbody, 0)
        pltpu.sync_copy(ex_st, ex_hbm.at[pl.ds(off, CHA)])
        pltpu.async_copy(exf_st, den_sh.at[d8_ch], sem_s, add=True).wait()
        return 0

    lax.fori_loop(0, NCHA, chunk_body, 0)
    plsc.subcore_barrier()
    pltpu.sync_copy(den_sh.at[pl.ds(si * NDT, NDT)],
                    den_hbm.at[pl.ds(ci * ND + si * NDT, NDT)])


@functools.partial(
    pl.kernel,
    out_type=[jax.ShapeDtypeStruct((2 * NP, F), _f32)],
    mesh=_mesh,
    scratch_types=[
        pltpu.VMEM((CH,), _i32),
        pltpu.VMEM((CH,), _i32),
        pltpu.VMEM((CH,), _i32),
        pltpu.VMEM((CH,), _f32),
        pltpu.VMEM((CH, F), _f32),
        pltpu.VMEM((CH, 16), _f32),
        pltpu.VMEM((CH, F), _f32),
        pltpu.VMEM((CH, F), _f32),
        pltpu.VMEM_SHARED((NP, F), _f32),
        pltpu.SemaphoreType.DMA,
        pltpu.SemaphoreType.DMA,
    ],
)
def _sc_aggregate(hl, ex_hbm, dinv_hbm, s_hbm, d_hbm, d8_hbm, dmf_hbm,
                  zf_hbm,
                  out_hbm,
                  s_ch, d_ch, d8_ch, dmf_ch, sbuf, ex_st, dvb, mbuf, out_sh,
                  sem_g, sem_s):
    ci = lax.axis_index("c")
    si = lax.axis_index("s")
    wid = ci * 16 + si
    base = wid * EPT
    pltpu.sync_copy(zf_hbm, out_sh.at[pl.ds(si * NPT, NPT)])
    plsc.subcore_barrier()

    def chunk_body(cc, _):
        off = base + cc * CH
        pltpu.sync_copy(s_hbm.at[pl.ds(off, CH)], s_ch)
        pltpu.sync_copy(d_hbm.at[pl.ds(off, CH)], d_ch)
        pltpu.sync_copy(d8_hbm.at[pl.ds(off, CH)], d8_ch)
        pltpu.sync_copy(dmf_hbm.at[pl.ds(off, CH)], dmf_ch)
        pltpu.sync_copy(ex_hbm.at[pl.ds(off, CH)], ex_st)
        pltpu.async_copy(hl.at[s_ch], sbuf, sem_g).wait()
        pltpu.async_copy(dinv_hbm.at[d8_ch], dvb, sem_g).wait()

        def gbody(g, _g):
            dmfv = dmf_ch[pl.ds(g * 16, 16)]

            def ebody(j, _e):
                e = g * 16 + j
                dmb = _bcast_lane(dmfv, j)
                dv16 = jnp.zeros((16,), _f32)
                for j8 in range(8):
                    msk = jnp.maximum(0.0, 1.0 - jnp.abs(dmb - float(j8)))
                    dv16 = dv16 + dvb[e, pl.ds(j8 * 16, 16)] * msk
                w = ex_st[e] * dv16
                for h in range(H):
                    wb = _bcast_lane(w, h)
                    mbuf[e, pl.ds(h * 16, 16)] = (
                        wb * sbuf[e, pl.ds(h * 16, 16)])
                return 0

            lax.fori_loop(0, 16, ebody, 0)
            return 0

        lax.fori_loop(0, CH // 16, gbody, 0)
        pltpu.async_copy(mbuf, out_sh.at[d_ch], sem_s, add=True).wait()
        return 0

    lax.fori_loop(0, NCH, chunk_body, 0)
    plsc.subcore_barrier()
    pltpu.sync_copy(out_sh.at[pl.ds(si * NPT, NPT)],
                    out_hbm.at[pl.ds(ci * NP + si * NPT, NPT)])


def _dot(a, b):
    return jnp.dot(a, b, precision=lax.Precision.HIGHEST,
                   preferred_element_type=_f32)


def _prep_body(x_ref, ea_ref, wne_ref, bne_ref, wl_ref, bl_ref, wr_ref,
               br_ref, ea2_ref, hl_ref, hr_ref):
    h0 = jax.nn.relu(_dot(x_ref[...], wne_ref[...]) + bne_ref[...])
    hl_ref[...] = _dot(h0, wl_ref[...]) + bl_ref[...]
    hr_ref[...] = _dot(h0, wr_ref[...]) + br_ref[...]
    mean = jnp.sum(ea_ref[...]) / E
    ea2_ref[0:2500, :] = ea_ref[...]
    gid = (E + lax.broadcasted_iota(_i32, (92, F), 0) * F
           + lax.broadcasted_iota(_i32, (92, F), 1))
    ea2_ref[2500:2592, :] = jnp.where(gid < NE, mean, 0.0)


_prep = pl.pallas_call(
    _prep_body,
    out_shape=[
        jax.ShapeDtypeStruct((2592, F), _f32),
        jax.ShapeDtypeStruct((N, F), _f32),
        jax.ShapeDtypeStruct((N, F), _f32),
    ],
)


def _mid_body(den_ref, dinv_ref):
    dinv_ref[...] = 1.0 / (den_ref[0] + den_ref[1] + 1e-16)


_mid = pl.pallas_call(
    _mid_body,
    out_shape=[jax.ShapeDtypeStruct((ND, F), _f32)],
)


def _post_body(has_resid, has_next, *refs):
    if has_resid:
        (op_ref, bias_ref, g_ref, b_ref, res_ref, *rest) = refs
    else:
        (op_ref, bias_ref, g_ref, b_ref, *rest) = refs
    if has_next:
        (wl_ref, bl_ref, wr_ref, br_ref, h_ref, hl_ref, hr_ref) = rest
    else:
        (h_ref,) = rest
    y = op_ref[0, 0:N, :] + op_ref[1, 0:N, :] + bias_ref[...]
    mu = jnp.mean(y, axis=0)
    d = y - mu
    var = jnp.mean(d * d, axis=0)
    z = d * lax.rsqrt(var + 1e-5) * g_ref[...] + b_ref[...]
    if has_resid:
        z = z + res_ref[...]
    h = jax.nn.relu(z)
    h_ref[...] = h
    if has_next:
        hl_ref[...] = _dot(h, wl_ref[...]) + bl_ref[...]
        hr_ref[...] = _dot(h, wr_ref[...]) + br_ref[...]


def _make_post(has_resid, has_next):
    outs = [jax.ShapeDtypeStruct((N, F), _f32)]
    if has_next:
        outs = outs + [jax.ShapeDtypeStruct((N, F), _f32),
                       jax.ShapeDtypeStruct((N, F), _f32)]
    return pl.pallas_call(
        functools.partial(_post_body, has_resid, has_next), out_shape=outs)


_post_first = _make_post(False, True)
_post_midl = _make_post(True, True)
_post_last = _make_post(True, False)


def _head_body(h_ref, batch_ref, m_in_ref, g_in_ref, p_in_ref, shap_ref,
               wm1, bm1, wm2, bm2, wg1, bg1, wg2, bg2, wp1, bp1, wp2, bp2,
               ws, bs, w1p, w1m, w1g, w1pc, w1s, b1, w2, b2, w3, b3,
               out_ref):
    onehot = jnp.where(
        lax.broadcasted_iota(_i32, (B, N), 0) == batch_ref[...], 1.0, 0.0)
    cnt = jnp.sum(onehot, axis=1, keepdims=True)
    pooled = _dot(onehot, h_ref[...]) / jnp.maximum(cnt, 1.0)
    meta = jax.nn.relu(_dot(jax.nn.relu(
        _dot(m_in_ref[...], wm1[...]) + bm1[...]), wm2[...]) + bm2[...])
    gre = jax.nn.relu(_dot(jax.nn.relu(
        _dot(g_in_ref[...], wg1[...]) + bg1[...]), wg2[...]) + bg2[...])
    pca = jax.nn.relu(_dot(jax.nn.relu(
        _dot(p_in_ref[...], wp1[...]) + bp1[...]), wp2[...]) + bp2[...])
    xs = jax.nn.relu(_dot(shap_ref[...], ws[...]) + bs[...])
    z = (_dot(pooled, w1p[...]) + _dot(meta, w1m[...]) + _dot(gre, w1g[...])
         + _dot(pca, w1pc[...]) + _dot(xs, w1s[...]) + b1[...])
    z = jax.nn.relu(z)
    z = jax.nn.relu(_dot(z, w2[...]) + b2[...])
    out_ref[...] = _dot(z, w3[...]) + b3[...]


_head = pl.pallas_call(
    _head_body, out_shape=[jax.ShapeDtypeStruct((B, 1), _f32)])


def _gat_layer(hl, hr, edges, p, consts):
    s_all, d_all, d8_all, ea2, dmf_all = edges
    oh, kh, zd, zf = consts
    we_r = p["We"][0].reshape(H, 16)
    att_r = p["att"]
    ex, den = _sc_logits(hl, hr, s_all, d_all, d8_all, ea2, dmf_all,
                         we_r, att_r, oh, kh, zd)
    dinv, = _mid(den.reshape(2, ND, F))
    out_parts, = _sc_aggregate(hl, ex, dinv, s_all, d_all, d8_all, dmf_all,
                               zf)
    return out_parts.reshape(2, NP, F)


def kernel(x, edge_index, edge_attr, batch, global_features, shap_embedding,
           params):
    p = params
    src = edge_index[0].astype(_i32)
    dst = edge_index[1].astype(_i32)
    loop = jnp.arange(N, dtype=_i32)
    padz = jnp.zeros((EP - NE,), _i32)
    s_all = jnp.concatenate([src, loop, padz])
    d_all = jnp.concatenate([dst, loop, padz])
    d8_all = d_all // 8
    dmf_all = (d_all % 8).astype(_f32)
    ea_r = edge_attr[:, 0].reshape(2500, F)
    zd = jnp.zeros((NDT, F), _f32)
    zf = jnp.zeros((NPT, F), _f32)
    oh = jnp.eye(H, 16, dtype=_f32)
    kh = jnp.broadcast_to(
        (jnp.arange(16) < H).astype(_f32)[None, :], (16, 16))

    ea2_r, hl, hr = _prep(x, ea_r, p["ne"]["W"], p["ne"]["b"],
                          p["g1"]["Wl"], p["g1"]["bl"],
                          p["g1"]["Wr"], p["g1"]["br"])
    ea2 = ea2_r.reshape(EP)
    edges = (s_all, d_all, d8_all, ea2, dmf_all)
    consts = (oh, kh, zd, zf)

    op = _gat_layer(hl, hr, edges, p["g1"], consts)
    h1, hl, hr = _post_first(op, p["g1"]["bias"], p["bn1"]["g"],
                             p["bn1"]["b"],
                             p["g2"]["Wl"], p["g2"]["bl"],
                             p["g2"]["Wr"], p["g2"]["br"])
    op = _gat_layer(hl, hr, edges, p["g2"], consts)
    h2, hl, hr = _post_midl(op, p["g2"]["bias"], p["bn2"]["g"], p["bn2"]["b"],
                            h1, p["g3"]["Wl"], p["g3"]["bl"],
                            p["g3"]["Wr"], p["g3"]["br"])
    op = _gat_layer(hl, hr, edges, p["g3"], consts)
    h3, hl, hr = _post_midl(op, p["g3"]["bias"], p["bn3"]["g"], p["bn3"]["b"],
                            h2, p["g4"]["Wl"], p["g4"]["bl"],
                            p["g4"]["Wr"], p["g4"]["br"])
    op = _gat_layer(hl, hr, edges, p["g4"], consts)
    h4, = _post_last(op, p["g4"]["bias"], p["bn4"]["g"], p["bn4"]["b"], h3)

    gf = global_features[:, 0, :]
    w1 = p["fc1"]["W"]
    out, = _head(h4, batch.astype(_i32).reshape(1, N),
                 gf[:, 0:4], gf[:, 4:6], gf[:, 6:16], shap_embedding,
                 p["meta1"]["W"], p["meta1"]["b"], p["meta2"]["W"],
                 p["meta2"]["b"], p["gr1"]["W"], p["gr1"]["b"],
                 p["gr2"]["W"], p["gr2"]["b"], p["pca1"]["W"], p["pca1"]["b"],
                 p["pca2"]["W"], p["pca2"]["b"], p["shap"]["W"],
                 p["shap"]["b"], w1[0:128], w1[128:144], w1[144:160],
                 w1[160:192], w1[192:224], p["fc1"]["b"],
                 p["fc2"]["W"], p["fc2"]["b"], p["fc3"]["W"], p["fc3"]["b"])
    return out


# speedup vs baseline: 17.7133x; 1.0501x over previous
import functools

import jax
import jax.numpy as jnp
from jax import lax
from jax.experimental import pallas as pl
from jax.experimental.pallas import tpu as pltpu
from jax.experimental.pallas import tpu_sc as plsc

N = 10000
E = 320000
B = 64
H = 8
C = 16
F = H * C

NE = E + N
NT = 32
CH = 64
EP = 331776
EPT = EP // NT
NCH = EPT // CH
CHA = 128
NCHA = EPT // CHA
NP = 10240
NPT = NP // 16
ND = NP // 8
NDT = ND // 16

_mesh = plsc.VectorSubcoreMesh(core_axis_name="c", subcore_axis_name="s")
_f32 = jnp.float32
_i32 = jnp.int32

_GDN = lax.GatherDimensionNumbers(
    offset_dims=(), collapsed_slice_dims=(0,), start_index_map=(0,))


def _perm(v, idx):
    return lax.gather(v, idx[:, None], _GDN, (1,),
                      mode=lax.GatherScatterMode.PROMISE_IN_BOUNDS)


def _bcast_lane(v, lane):
    return _perm(v, jnp.full((16,), lane, _i32))


def _lanesum(v, rots):
    for r in rots:
        v = v + _perm(v, r)
    return v


@functools.partial(
    pl.kernel,
    out_type=[
        jax.ShapeDtypeStruct((EP, 16), _f32),
        jax.ShapeDtypeStruct((2 * ND, F), _f32),
    ],
    mesh=_mesh,
    scratch_types=[
        pltpu.VMEM((CHA,), _i32),
        pltpu.VMEM((CHA,), _i32),
        pltpu.VMEM((CHA,), _i32),
        pltpu.VMEM((CHA,), _f32),
        pltpu.VMEM((CHA,), _f32),
        pltpu.VMEM((H, 16), _f32),
        pltpu.VMEM((H, 16), _f32),
        pltpu.VMEM((H, 16), _f32),
        pltpu.VMEM((16, 16), _f32),
        pltpu.VMEM((CHA, F), _f32),
        pltpu.VMEM((CHA, F), _f32),
        pltpu.VMEM((CHA, 16), _f32),
        pltpu.VMEM((CHA, F), _f32),
        pltpu.VMEM_SHARED((ND, F), _f32),
        pltpu.SemaphoreType.DMA,
        pltpu.SemaphoreType.DMA,
    ],
)
def _sc_logits(hl, hr, s_hbm, d_hbm, d8_hbm, ea_hbm, dmf_hbm, web_hbm,
               attb_hbm, oh_hbm, kh_hbm, zd_hbm,
               ex_hbm, den_hbm,
               s_ch, d_ch, d8_ch, ea_ch, dmf_ch, web_vm, attb_vm, oh_vm,
               kh_vm, sbuf, rbuf, ex_st, exf_st, den_sh, sem_g, sem_s):
    ci = lax.axis_index("c")
    si = lax.axis_index("s")
    wid = ci * 16 + si
    base = wid * EPT
    pltpu.sync_copy(web_hbm, web_vm)
    pltpu.sync_copy(attb_hbm, attb_vm)
    pltpu.sync_copy(oh_hbm, oh_vm)
    pltpu.sync_copy(kh_hbm, kh_vm)
    pltpu.sync_copy(zd_hbm, den_sh.at[pl.ds(si * NDT, NDT)])
    plsc.subcore_barrier()

    it16 = lax.iota(_i32, 16)
    rots = [(it16 + r) & 15 for r in (8, 4, 2, 1)]
    webs = [web_vm[h] for h in range(H)]
    atts = [attb_vm[h] for h in range(H)]
    ohs = [oh_vm[h] for h in range(H)]
    khdr = kh_vm[0]

    def chunk_body(cc, _):
        off = base + cc * CHA
        pltpu.sync_copy(s_hbm.at[pl.ds(off, CHA)], s_ch)
        pltpu.sync_copy(d_hbm.at[pl.ds(off, CHA)], d_ch)
        pltpu.sync_copy(d8_hbm.at[pl.ds(off, CHA)], d8_ch)
        pltpu.sync_copy(ea_hbm.at[pl.ds(off, CHA)], ea_ch)
        pltpu.sync_copy(dmf_hbm.at[pl.ds(off, CHA)], dmf_ch)
        pltpu.async_copy(hl.at[s_ch], sbuf, sem_g).wait()
        pltpu.async_copy(hr.at[d_ch], rbuf, sem_g).wait()

        def gbody(g, _g):
            eav = ea_ch[pl.ds(g * 16, 16)]
            dmfv = dmf_ch[pl.ds(g * 16, 16)]

            def ebody(j, _e):
                e = g * 16 + j
                eab = _bcast_lane(eav, j)
                dmb = _bcast_lane(dmfv, j)
                acc = jnp.zeros((16,), _f32)
                for h in range(H):
                    u = (sbuf[e, pl.ds(h * 16, 16)]
                         + rbuf[e, pl.ds(h * 16, 16)] + eab * webs[h])
                    u = jnp.maximum(u, 0.2 * u)
                    tot = _lanesum(u * atts[h], rots)
                    acc = acc + tot * ohs[h]
                exv = jnp.exp(acc)
                livef = jnp.where((off + e) < NE, 1.0, 0.0)
                keep = khdr * lax.broadcast_in_dim(livef, (16,), ())
                exv = exv * keep
                ex_st[e] = exv
                for j8 in range(8):
                    msk = jnp.maximum(0.0, 1.0 - jnp.abs(dmb - float(j8)))
                    exf_st[e, pl.ds(j8 * 16, 16)] = exv * msk
                return 0

            lax.fori_loop(0, 16, ebody, 0, unroll=2)
            return 0

        lax.fori_loop(0, CHA // 16, gbody, 0)
        pltpu.sync_copy(ex_st, ex_hbm.at[pl.ds(off, CHA)])
        pltpu.async_copy(exf_st, den_sh.at[d8_ch], sem_s, add=True).wait()
        return 0

    lax.fori_loop(0, NCHA, chunk_body, 0)
    plsc.subcore_barrier()
    pltpu.sync_copy(den_sh.at[pl.ds(si * NDT, NDT)],
                    den_hbm.at[pl.ds(ci * ND + si * NDT, NDT)])


@functools.partial(
    pl.kernel,
    out_type=[jax.ShapeDtypeStruct((2 * NP, F), _f32)],
    mesh=_mesh,
    scratch_types=[
        pltpu.VMEM((CH,), _i32),
        pltpu.VMEM((CH,), _i32),
        pltpu.VMEM((CH,), _i32),
        pltpu.VMEM((CH,), _f32),
        pltpu.VMEM((CH, F), _f32),
        pltpu.VMEM((CH, 16), _f32),
        pltpu.VMEM((CH, F), _f32),
        pltpu.VMEM((CH, F), _f32),
        pltpu.VMEM_SHARED((NP, F), _f32),
        pltpu.SemaphoreType.DMA,
        pltpu.SemaphoreType.DMA,
    ],
)
def _sc_aggregate(hl, ex_hbm, dinv_hbm, s_hbm, d_hbm, d8_hbm, dmf_hbm,
                  zf_hbm,
                  out_hbm,
                  s_ch, d_ch, d8_ch, dmf_ch, sbuf, ex_st, dvb, mbuf, out_sh,
                  sem_g, sem_s):
    ci = lax.axis_index("c")
    si = lax.axis_index("s")
    wid = ci * 16 + si
    base = wid * EPT
    pltpu.sync_copy(zf_hbm, out_sh.at[pl.ds(si * NPT, NPT)])
    plsc.subcore_barrier()

    def chunk_body(cc, _):
        off = base + cc * CH
        pltpu.sync_copy(s_hbm.at[pl.ds(off, CH)], s_ch)
        pltpu.sync_copy(d_hbm.at[pl.ds(off, CH)], d_ch)
        pltpu.sync_copy(d8_hbm.at[pl.ds(off, CH)], d8_ch)
        pltpu.sync_copy(dmf_hbm.at[pl.ds(off, CH)], dmf_ch)
        pltpu.sync_copy(ex_hbm.at[pl.ds(off, CH)], ex_st)
        pltpu.async_copy(hl.at[s_ch], sbuf, sem_g).wait()
        pltpu.async_copy(dinv_hbm.at[d8_ch], dvb, sem_g).wait()

        def gbody(g, _g):
            dmfv = dmf_ch[pl.ds(g * 16, 16)]

            def ebody(j, _e):
                e = g * 16 + j
                dmb = _bcast_lane(dmfv, j)
                dv16 = jnp.zeros((16,), _f32)
                for j8 in range(8):
                    msk = jnp.maximum(0.0, 1.0 - jnp.abs(dmb - float(j8)))
                    dv16 = dv16 + dvb[e, pl.ds(j8 * 16, 16)] * msk
                w = ex_st[e] * dv16
                for h in range(H):
                    wb = _bcast_lane(w, h)
                    mbuf[e, pl.ds(h * 16, 16)] = (
                        wb * sbuf[e, pl.ds(h * 16, 16)])
                return 0

            lax.fori_loop(0, 16, ebody, 0, unroll=2)
            return 0

        lax.fori_loop(0, CH // 16, gbody, 0)
        pltpu.async_copy(mbuf, out_sh.at[d_ch], sem_s, add=True).wait()
        return 0

    lax.fori_loop(0, NCH, chunk_body, 0)
    plsc.subcore_barrier()
    pltpu.sync_copy(out_sh.at[pl.ds(si * NPT, NPT)],
                    out_hbm.at[pl.ds(ci * NP + si * NPT, NPT)])


def _dot(a, b):
    return jnp.dot(a, b, precision=lax.Precision.HIGHEST,
                   preferred_element_type=_f32)


def _prep_body(x_ref, ea_ref, wne_ref, bne_ref, wl_ref, bl_ref, wr_ref,
               br_ref, ea2_ref, hl_ref, hr_ref):
    h0 = jax.nn.relu(_dot(x_ref[...], wne_ref[...]) + bne_ref[...])
    hl_ref[...] = _dot(h0, wl_ref[...]) + bl_ref[...]
    hr_ref[...] = _dot(h0, wr_ref[...]) + br_ref[...]
    mean = jnp.sum(ea_ref[...]) / E
    ea2_ref[0:2500, :] = ea_ref[...]
    gid = (E + lax.broadcasted_iota(_i32, (92, F), 0) * F
           + lax.broadcasted_iota(_i32, (92, F), 1))
    ea2_ref[2500:2592, :] = jnp.where(gid < NE, mean, 0.0)


_prep = pl.pallas_call(
    _prep_body,
    out_shape=[
        jax.ShapeDtypeStruct((2592, F), _f32),
        jax.ShapeDtypeStruct((N, F), _f32),
        jax.ShapeDtypeStruct((N, F), _f32),
    ],
)


def _mid_body(den_ref, dinv_ref):
    dinv_ref[...] = 1.0 / (den_ref[0] + den_ref[1] + 1e-16)


_mid = pl.pallas_call(
    _mid_body,
    out_shape=[jax.ShapeDtypeStruct((ND, F), _f32)],
)


def _post_body(has_resid, has_next, *refs):
    if has_resid:
        (op_ref, bias_ref, g_ref, b_ref, res_ref, *rest) = refs
    else:
        (op_ref, bias_ref, g_ref, b_ref, *rest) = refs
    if has_next:
        (wl_ref, bl_ref, wr_ref, br_ref, h_ref, hl_ref, hr_ref) = rest
    else:
        (h_ref,) = rest
    y = op_ref[0, 0:N, :] + op_ref[1, 0:N, :] + bias_ref[...]
    mu = jnp.mean(y, axis=0)
    d = y - mu
    var = jnp.mean(d * d, axis=0)
    z = d * lax.rsqrt(var + 1e-5) * g_ref[...] + b_ref[...]
    if has_resid:
        z = z + res_ref[...]
    h = jax.nn.relu(z)
    h_ref[...] = h
    if has_next:
        hl_ref[...] = _dot(h, wl_ref[...]) + bl_ref[...]
        hr_ref[...] = _dot(h, wr_ref[...]) + br_ref[...]


def _make_post(has_resid, has_next):
    outs = [jax.ShapeDtypeStruct((N, F), _f32)]
    if has_next:
        outs = outs + [jax.ShapeDtypeStruct((N, F), _f32),
                       jax.ShapeDtypeStruct((N, F), _f32)]
    return pl.pallas_call(
        functools.partial(_post_body, has_resid, has_next), out_shape=outs)


_post_first = _make_post(False, True)
_post_midl = _make_post(True, True)
_post_last = _make_post(True, False)


def _head_body(h_ref, batch_ref, m_in_ref, g_in_ref, p_in_ref, shap_ref,
               wm1, bm1, wm2, bm2, wg1, bg1, wg2, bg2, wp1, bp1, wp2, bp2,
               ws, bs, w1p, w1m, w1g, w1pc, w1s, b1, w2, b2, w3, b3,
               out_ref):
    onehot = jnp.where(
        lax.broadcasted_iota(_i32, (B, N), 0) == batch_ref[...], 1.0, 0.0)
    cnt = jnp.sum(onehot, axis=1, keepdims=True)
    pooled = _dot(onehot, h_ref[...]) / jnp.maximum(cnt, 1.0)
    meta = jax.nn.relu(_dot(jax.nn.relu(
        _dot(m_in_ref[...], wm1[...]) + bm1[...]), wm2[...]) + bm2[...])
    gre = jax.nn.relu(_dot(jax.nn.relu(
        _dot(g_in_ref[...], wg1[...]) + bg1[...]), wg2[...]) + bg2[...])
    pca = jax.nn.relu(_dot(jax.nn.relu(
        _dot(p_in_ref[...], wp1[...]) + bp1[...]), wp2[...]) + bp2[...])
    xs = jax.nn.relu(_dot(shap_ref[...], ws[...]) + bs[...])
    z = (_dot(pooled, w1p[...]) + _dot(meta, w1m[...]) + _dot(gre, w1g[...])
         + _dot(pca, w1pc[...]) + _dot(xs, w1s[...]) + b1[...])
    z = jax.nn.relu(z)
    z = jax.nn.relu(_dot(z, w2[...]) + b2[...])
    out_ref[...] = _dot(z, w3[...]) + b3[...]


_head = pl.pallas_call(
    _head_body, out_shape=[jax.ShapeDtypeStruct((B, 1), _f32)])


def _gat_layer(hl, hr, edges, p, consts):
    s_all, d_all, d8_all, ea2, dmf_all = edges
    oh, kh, zd, zf = consts
    we_r = p["We"][0].reshape(H, 16)
    att_r = p["att"]
    ex, den = _sc_logits(hl, hr, s_all, d_all, d8_all, ea2, dmf_all,
                         we_r, att_r, oh, kh, zd)
    dinv, = _mid(den.reshape(2, ND, F))
    out_parts, = _sc_aggregate(hl, ex, dinv, s_all, d_all, d8_all, dmf_all,
                               zf)
    return out_parts.reshape(2, NP, F)


def kernel(x, edge_index, edge_attr, batch, global_features, shap_embedding,
           params):
    p = params
    src = edge_index[0].astype(_i32)
    dst = edge_index[1].astype(_i32)
    loop = jnp.arange(N, dtype=_i32)
    padz = jnp.zeros((EP - NE,), _i32)
    s_all = jnp.concatenate([src, loop, padz])
    d_all = jnp.concatenate([dst, loop, padz])
    d8_all = d_all // 8
    dmf_all = (d_all % 8).astype(_f32)
    ea_r = edge_attr[:, 0].reshape(2500, F)
    zd = jnp.zeros((NDT, F), _f32)
    zf = jnp.zeros((NPT, F), _f32)
    oh = jnp.eye(H, 16, dtype=_f32)
    kh = jnp.broadcast_to(
        (jnp.arange(16) < H).astype(_f32)[None, :], (16, 16))

    ea2_r, hl, hr = _prep(x, ea_r, p["ne"]["W"], p["ne"]["b"],
                          p["g1"]["Wl"], p["g1"]["bl"],
                          p["g1"]["Wr"], p["g1"]["br"])
    ea2 = ea2_r.reshape(EP)
    edges = (s_all, d_all, d8_all, ea2, dmf_all)
    consts = (oh, kh, zd, zf)

    op = _gat_layer(hl, hr, edges, p["g1"], consts)
    h1, hl, hr = _post_first(op, p["g1"]["bias"], p["bn1"]["g"],
                             p["bn1"]["b"],
                             p["g2"]["Wl"], p["g2"]["bl"],
                             p["g2"]["Wr"], p["g2"]["br"])
    op = _gat_layer(hl, hr, edges, p["g2"], consts)
    h2, hl, hr = _post_midl(op, p["g2"]["bias"], p["bn2"]["g"], p["bn2"]["b"],
                            h1, p["g3"]["Wl"], p["g3"]["bl"],
                            p["g3"]["Wr"], p["g3"]["br"])
    op = _gat_layer(hl, hr, edges, p["g3"], consts)
    h3, hl, hr = _post_midl(op, p["g3"]["bias"], p["bn3"]["g"], p["bn3"]["b"],
                            h2, p["g4"]["Wl"], p["g4"]["bl"],
                            p["g4"]["Wr"], p["g4"]["br"])
    op = _gat_layer(hl, hr, edges, p["g4"], consts)
    h4, = _post_last(op, p["g4"]["bias"], p["bn4"]["g"], p["bn4"]["b"], h3)

    gf = global_features[:, 0, :]
    w1 = p["fc1"]["W"]
    out, = _head(h4, batch.astype(_i32).reshape(1, N),
                 gf[:, 0:4], gf[:, 4:6], gf[:, 6:16], shap_embedding,
                 p["meta1"]["W"], p["meta1"]["b"], p["meta2"]["W"],
                 p["meta2"]["b"], p["gr1"]["W"], p["gr1"]["b"],
                 p["gr2"]["W"], p["gr2"]["b"], p["pca1"]["W"], p["pca1"]["b"],
                 p["pca2"]["W"], p["pca2"]["b"], p["shap"]["W"],
                 p["shap"]["b"], w1[0:128], w1[128:144], w1[144:160],
                 w1[160:192], w1[192:224], p["fc1"]["b"],
                 p["fc2"]["W"], p["fc2"]["b"], p["fc3"]["W"], p["fc3"]["b"])
    return out
